# baseline (speedup 1.0000x reference)
import functools

import jax
import jax.numpy as jnp
from jax import lax
from jax.experimental import pallas as pl
from jax.experimental.pallas import tpu as pltpu
from jax.experimental.pallas import tpu_sc as plsc

N_WORDS = 100000
EMBED_DIM = 300
PAD_DIM = 304
BATCH = 16384

_info = plsc.get_sparse_core_info()
_NC = _info.num_cores
_NS = _info.num_subcores
_NW = _NC * _NS
_BPW = BATCH // _NW
_CHUNK = 128
_NCHUNK = _BPW // _CHUNK
_NT = 3 * _NCHUNK

_mesh = plsc.VectorSubcoreMesh(core_axis_name="c", subcore_axis_name="s")

_RPT = N_WORDS // _NW


@functools.partial(
    pl.kernel,
    mesh=_mesh,
    compiler_params=pltpu.CompilerParams(use_tc_tiling_on_sc=False),
    out_type=jax.ShapeDtypeStruct((N_WORDS, PAD_DIM), jnp.float32),
    scratch_types=[
        pltpu.SemaphoreType.DMA,
        pltpu.SemaphoreType.DMA,
    ],
)
def _pad_sc(table_hbm, tail8_hbm, pad_hbm, sem_a, sem_b):
    wid = lax.axis_index("s") * _NC + lax.axis_index("c")
    r0 = wid * _RPT
    a = pltpu.make_async_copy(
        table_hbm.at[pl.ds(r0, _RPT), pl.ds(0, 296)],
        pad_hbm.at[pl.ds(r0, _RPT), pl.ds(0, 296)],
        sem_a,
    )
    a.start()
    b = pltpu.make_async_copy(
        tail8_hbm.at[pl.ds(r0, _RPT)],
        pad_hbm.at[pl.ds(r0, _RPT), pl.ds(296, 8)],
        sem_b,
    )
    b.start()
    a.wait()
    b.wait()


@functools.partial(
    pl.kernel,
    mesh=_mesh,
    compiler_params=pltpu.CompilerParams(use_tc_tiling_on_sc=False),
    out_type=[jax.ShapeDtypeStruct((BATCH, EMBED_DIM), jnp.float32)] * 3,
    scratch_types=[
        pltpu.VMEM((_NT, _CHUNK), jnp.int32),
        pltpu.VMEM((_CHUNK, PAD_DIM), jnp.float32),
        pltpu.VMEM((_CHUNK, PAD_DIM), jnp.float32),
        pltpu.VMEM((_CHUNK, EMBED_DIM), jnp.float32),
        pltpu.SemaphoreType.DMA,
        pltpu.SemaphoreType.DMA,
        pltpu.SemaphoreType.DMA,
        pltpu.SemaphoreType.DMA,
    ],
)
def _embed3(tw_hbm, syn_hbm, ant_hbm, table_hbm, out_tw, out_syn, out_ant,
            idx_v, rows0, rows1, buf, sem0, sem1, semw1, semw2):
    wid = lax.axis_index("s") * _NC + lax.axis_index("c")
    base = wid * _BPW
    pltpu.sync_copy(tw_hbm.at[wid], idx_v.at[pl.ds(0, _NCHUNK)])
    pltpu.sync_copy(syn_hbm.at[wid], idx_v.at[pl.ds(_NCHUNK, _NCHUNK)])
    pltpu.sync_copy(ant_hbm.at[wid], idx_v.at[pl.ds(2 * _NCHUNK, _NCHUNK)])
    outs = (out_tw, out_syn, out_ant)
    rows = (rows0, rows1)
    sems = (sem0, sem1)

    def fire(t):
        cp = pltpu.make_async_copy(
            table_hbm.at[idx_v.at[t]], rows[t % 2], sems[t % 2]
        )
        cp.start()
        return cp

    cp = fire(0)
    for t in range(_NT):
        cp.wait()
        if t + 1 < _NT:
            nxt = fire(t + 1)
        src = rows[t % 2]
        out_hbm = outs[t // _NCHUNK]
        off = base + (t % _NCHUNK) * _CHUNK

        @pl.loop(0, _CHUNK)
        def _repack(k):
            for j in range(18):
                buf[k, pl.ds(j * 16, 16)] = src[k, pl.ds(j * 16, 16)]
            buf[k, pl.ds(284, 16)] = src[k, pl.ds(284, 16)]

        pltpu.sync_copy(buf, out_hbm.at[pl.ds(off, _CHUNK)])
        if t + 1 < _NT:
            cp = nxt


def kernel(target_word, synonym, antonym, embedding_weight):
    tw = target_word.astype(jnp.int32).reshape(_NW, _NCHUNK, _CHUNK)
    syn = synonym.astype(jnp.int32).reshape(_NW, _NCHUNK, _CHUNK)
    ant = antonym.astype(jnp.int32).reshape(_NW, _NCHUNK, _CHUNK)
    tail8 = jnp.pad(embedding_weight[:, 296:EMBED_DIM], ((0, 0), (0, 4)))
    tab = _pad_sc(embedding_weight, tail8)
    o = _embed3(tw, syn, ant, tab)
    return (o[0], o[1], o[2])


# speedup vs baseline: 13.7458x; 13.7458x over previous
import functools

import jax
import jax.numpy as jnp
from jax import lax
from jax.experimental import pallas as pl
from jax.experimental.pallas import tpu as pltpu
from jax.experimental.pallas import tpu_sc as plsc

N_WORDS = 100000
EMBED_DIM = 300
PAD_DIM = 304
BATCH = 16384

_info = plsc.get_sparse_core_info()
_NC = _info.num_cores
_NS = _info.num_subcores
_NW = _NC * _NS
_BPW = BATCH // _NW
_CHUNK = 128
_NCHUNK = _BPW // _CHUNK
_NT = 3 * _NCHUNK

_mesh = plsc.VectorSubcoreMesh(core_axis_name="c", subcore_axis_name="s")

_RPT = N_WORDS // _NW


_PB = 125
_NPB = _RPT // _PB


@functools.partial(
    pl.kernel,
    mesh=_mesh,
    compiler_params=pltpu.CompilerParams(use_tc_tiling_on_sc=False),
    out_type=jax.ShapeDtypeStruct((N_WORDS, PAD_DIM), jnp.float32),
    scratch_types=[
        pltpu.VMEM((_PB, 296), jnp.float32),
        pltpu.VMEM((_PB, 296), jnp.float32),
        pltpu.VMEM((_PB, 8), jnp.float32),
        pltpu.VMEM((_PB, 8), jnp.float32),
        pltpu.SemaphoreType.DMA,
        pltpu.SemaphoreType.DMA,
        pltpu.SemaphoreType.DMA,
        pltpu.SemaphoreType.DMA,
    ],
)
def _pad_sc(table_hbm, tail8_hbm, pad_hbm,
            vin0, vin1, vt0, vt1, sem_r, sem_t, sem_w0, sem_w1):
    wid = lax.axis_index("s") * _NC + lax.axis_index("c")
    r0 = wid * _RPT
    vins = (vin0, vin1)
    vts = (vt0, vt1)
    semws = (sem_w0, sem_w1)

    def fire_read(b):
        rb = r0 + b * _PB
        ra = pltpu.make_async_copy(
            table_hbm.at[pl.ds(rb, _PB), pl.ds(0, 296)], vins[b % 2], sem_r
        )
        ra.start()
        rb_ = pltpu.make_async_copy(
            tail8_hbm.at[pl.ds(rb, _PB)], vts[b % 2], sem_t
        )
        rb_.start()
        return ra, rb_

    cps = fire_read(0)
    wr = [None, None]
    for b in range(_NPB):
        cps[0].wait()
        cps[1].wait()
        if b + 1 < _NPB:
            if wr[(b + 1) % 2] is not None:
                for w in wr[(b + 1) % 2]:
                    w.wait()
                wr[(b + 1) % 2] = None
            nxt = fire_read(b + 1)
        rb = r0 + b * _PB
        w1 = pltpu.make_async_copy(
            vins[b % 2], pad_hbm.at[pl.ds(rb, _PB), pl.ds(0, 296)],
            semws[b % 2],
        )
        w1.start()
        w2 = pltpu.make_async_copy(
            vts[b % 2], pad_hbm.at[pl.ds(rb, _PB), pl.ds(296, 8)],
            semws[b % 2],
        )
        w2.start()
        wr[b % 2] = (w1, w2)
        if b + 1 < _NPB:
            cps = nxt
    for pair in wr:
        if pair is not None:
            for w in pair:
                w.wait()


@functools.partial(
    pl.kernel,
    mesh=_mesh,
    compiler_params=pltpu.CompilerParams(use_tc_tiling_on_sc=False),
    out_type=[jax.ShapeDtypeStruct((BATCH, EMBED_DIM), jnp.float32)] * 3,
    scratch_types=[
        pltpu.VMEM((_NT, _CHUNK), jnp.int32),
        pltpu.VMEM((_CHUNK, PAD_DIM), jnp.float32),
        pltpu.VMEM((_CHUNK, PAD_DIM), jnp.float32),
        pltpu.VMEM((_CHUNK, EMBED_DIM), jnp.float32),
        pltpu.SemaphoreType.DMA,
        pltpu.SemaphoreType.DMA,
        pltpu.SemaphoreType.DMA,
        pltpu.SemaphoreType.DMA,
    ],
)
def _embed3(tw_hbm, syn_hbm, ant_hbm, table_hbm, out_tw, out_syn, out_ant,
            idx_v, rows0, rows1, buf, sem0, sem1, semw1, semw2):
    wid = lax.axis_index("s") * _NC + lax.axis_index("c")
    base = wid * _BPW
    pltpu.sync_copy(tw_hbm.at[wid], idx_v.at[pl.ds(0, _NCHUNK)])
    pltpu.sync_copy(syn_hbm.at[wid], idx_v.at[pl.ds(_NCHUNK, _NCHUNK)])
    pltpu.sync_copy(ant_hbm.at[wid], idx_v.at[pl.ds(2 * _NCHUNK, _NCHUNK)])
    outs = (out_tw, out_syn, out_ant)
    rows = (rows0, rows1)
    sems = (sem0, sem1)

    def fire(t):
        cp = pltpu.make_async_copy(
            table_hbm.at[idx_v.at[t]], rows[t % 2], sems[t % 2]
        )
        cp.start()
        return cp

    cp = fire(0)
    for t in range(_NT):
        cp.wait()
        if t + 1 < _NT:
            nxt = fire(t + 1)
        src = rows[t % 2]
        out_hbm = outs[t // _NCHUNK]
        off = base + (t % _NCHUNK) * _CHUNK

        @pl.loop(0, _CHUNK)
        def _repack(k):
            for j in range(18):
                buf[k, pl.ds(j * 16, 16)] = src[k, pl.ds(j * 16, 16)]
            buf[k, pl.ds(284, 16)] = src[k, pl.ds(284, 16)]

        pltpu.sync_copy(buf, out_hbm.at[pl.ds(off, _CHUNK)])
        if t + 1 < _NT:
            cp = nxt


def kernel(target_word, synonym, antonym, embedding_weight):
    tw = target_word.astype(jnp.int32).reshape(_NW, _NCHUNK, _CHUNK)
    syn = synonym.astype(jnp.int32).reshape(_NW, _NCHUNK, _CHUNK)
    ant = antonym.astype(jnp.int32).reshape(_NW, _NCHUNK, _CHUNK)
    tail8 = jnp.pad(embedding_weight[:, 296:EMBED_DIM], ((0, 0), (0, 4)))
    tab = _pad_sc(embedding_weight, tail8)
    o = _embed3(tw, syn, ant, tab)
    return (o[0], o[1], o[2])


# speedup vs baseline: 26.9864x; 1.9633x over previous
import functools

import jax
import jax.numpy as jnp
from jax import lax
from jax.experimental import pallas as pl
from jax.experimental.pallas import tpu as pltpu
from jax.experimental.pallas import tpu_sc as plsc

N_WORDS = 100000
EMBED_DIM = 300
PAD_DIM = 304
BATCH = 16384

_info = plsc.get_sparse_core_info()
_NC = _info.num_cores
_NS = _info.num_subcores
_NW = _NC * _NS
_BPW = BATCH // _NW
_CHUNK = 128
_NCHUNK = _BPW // _CHUNK
_NT = 3 * _NCHUNK

_mesh = plsc.VectorSubcoreMesh(core_axis_name="c", subcore_axis_name="s")

_PAD_ROWS = 2000


def _pad_body(x_ref, o_ref):
    o_ref[:, :EMBED_DIM] = x_ref[...]
    o_ref[:, EMBED_DIM:] = jnp.zeros(
        (_PAD_ROWS, PAD_DIM - EMBED_DIM), jnp.float32
    )


_pad_table = pl.pallas_call(
    _pad_body,
    grid=(N_WORDS // _PAD_ROWS,),
    in_specs=[pl.BlockSpec((_PAD_ROWS, EMBED_DIM), lambda i: (i, 0))],
    out_specs=pl.BlockSpec((_PAD_ROWS, PAD_DIM), lambda i: (i, 0)),
    out_shape=jax.ShapeDtypeStruct((N_WORDS, PAD_DIM), jnp.float32),
)


@functools.partial(
    pl.kernel,
    mesh=_mesh,
    compiler_params=pltpu.CompilerParams(use_tc_tiling_on_sc=False),
    out_type=[jax.ShapeDtypeStruct((BATCH, EMBED_DIM), jnp.float32)] * 3,
    scratch_types=[
        pltpu.VMEM((_NT, _CHUNK), jnp.int32),
        pltpu.VMEM((_CHUNK, PAD_DIM), jnp.float32),
        pltpu.VMEM((_CHUNK, PAD_DIM), jnp.float32),
        pltpu.VMEM((_CHUNK, EMBED_DIM), jnp.float32),
        pltpu.SemaphoreType.DMA,
        pltpu.SemaphoreType.DMA,
        pltpu.SemaphoreType.DMA,
        pltpu.SemaphoreType.DMA,
    ],
)
def _embed3(tw_hbm, syn_hbm, ant_hbm, table_hbm, out_tw, out_syn, out_ant,
            idx_v, rows0, rows1, buf, sem0, sem1, semw1, semw2):
    wid = lax.axis_index("s") * _NC + lax.axis_index("c")
    base = wid * _BPW
    pltpu.sync_copy(tw_hbm.at[wid], idx_v.at[pl.ds(0, _NCHUNK)])
    pltpu.sync_copy(syn_hbm.at[wid], idx_v.at[pl.ds(_NCHUNK, _NCHUNK)])
    pltpu.sync_copy(ant_hbm.at[wid], idx_v.at[pl.ds(2 * _NCHUNK, _NCHUNK)])
    outs = (out_tw, out_syn, out_ant)
    rows = (rows0, rows1)
    sems = (sem0, sem1)

    def fire(t):
        cp = pltpu.make_async_copy(
            table_hbm.at[idx_v.at[t]], rows[t % 2], sems[t % 2]
        )
        cp.start()
        return cp

    cp = fire(0)
    for t in range(_NT):
        cp.wait()
        if t + 1 < _NT:
            nxt = fire(t + 1)
        src = rows[t % 2]
        out_hbm = outs[t // _NCHUNK]
        off = base + (t % _NCHUNK) * _CHUNK

        @pl.loop(0, _CHUNK)
        def _repack(k):
            for j in range(18):
                buf[k, pl.ds(j * 16, 16)] = src[k, pl.ds(j * 16, 16)]
            buf[k, pl.ds(284, 16)] = src[k, pl.ds(284, 16)]

        pltpu.sync_copy(buf, out_hbm.at[pl.ds(off, _CHUNK)])
        if t + 1 < _NT:
            cp = nxt


def kernel(target_word, synonym, antonym, embedding_weight):
    tw = target_word.astype(jnp.int32).reshape(_NW, _NCHUNK, _CHUNK)
    syn = synonym.astype(jnp.int32).reshape(_NW, _NCHUNK, _CHUNK)
    ant = antonym.astype(jnp.int32).reshape(_NW, _NCHUNK, _CHUNK)
    tab = _pad_table(embedding_weight)
    o = _embed3(tw, syn, ant, tab)
    return (o[0], o[1], o[2])


# speedup vs baseline: 27.0540x; 1.0025x over previous
import functools

import jax
import jax.numpy as jnp
from jax import lax
from jax.experimental import pallas as pl
from jax.experimental.pallas import tpu as pltpu
from jax.experimental.pallas import tpu_sc as plsc

N_WORDS = 100000
EMBED_DIM = 300
PAD_DIM = 304
BATCH = 16384

_info = plsc.get_sparse_core_info()
_NC = _info.num_cores
_NS = _info.num_subcores
_NW = _NC * _NS
_BPW = BATCH // _NW
_CHUNK = 128
_NCHUNK = _BPW // _CHUNK
_NT = 3 * _NCHUNK

_mesh = plsc.VectorSubcoreMesh(core_axis_name="c", subcore_axis_name="s")

_PAD_ROWS = 2000


def _pad_body(x_ref, o_ref):
    o_ref[:, :256] = x_ref[:, :256]
    edge = jnp.concatenate(
        [x_ref[:, 256:EMBED_DIM],
         jnp.zeros((_PAD_ROWS, PAD_DIM - EMBED_DIM), jnp.float32)],
        axis=1,
    )
    o_ref[:, 256:] = edge


_pad_table = pl.pallas_call(
    _pad_body,
    grid=(N_WORDS // _PAD_ROWS,),
    in_specs=[pl.BlockSpec((_PAD_ROWS, EMBED_DIM), lambda i: (i, 0))],
    out_specs=pl.BlockSpec((_PAD_ROWS, PAD_DIM), lambda i: (i, 0)),
    out_shape=jax.ShapeDtypeStruct((N_WORDS, PAD_DIM), jnp.float32),
)


@functools.partial(
    pl.kernel,
    mesh=_mesh,
    compiler_params=pltpu.CompilerParams(use_tc_tiling_on_sc=False),
    out_type=[jax.ShapeDtypeStruct((BATCH, EMBED_DIM), jnp.float32)] * 3,
    scratch_types=[
        pltpu.VMEM((_NT, _CHUNK), jnp.int32),
        pltpu.VMEM((_CHUNK, PAD_DIM), jnp.float32),
        pltpu.VMEM((_CHUNK, PAD_DIM), jnp.float32),
        pltpu.VMEM((_CHUNK, EMBED_DIM), jnp.float32),
        pltpu.SemaphoreType.DMA,
        pltpu.SemaphoreType.DMA,
        pltpu.SemaphoreType.DMA,
        pltpu.SemaphoreType.DMA,
    ],
)
def _embed3(tw_hbm, syn_hbm, ant_hbm, table_hbm, out_tw, out_syn, out_ant,
            idx_v, rows0, rows1, buf, sem0, sem1, semw1, semw2):
    wid = lax.axis_index("s") * _NC + lax.axis_index("c")
    base = wid * _BPW
    pltpu.sync_copy(tw_hbm.at[wid], idx_v.at[pl.ds(0, _NCHUNK)])
    pltpu.sync_copy(syn_hbm.at[wid], idx_v.at[pl.ds(_NCHUNK, _NCHUNK)])
    pltpu.sync_copy(ant_hbm.at[wid], idx_v.at[pl.ds(2 * _NCHUNK, _NCHUNK)])
    outs = (out_tw, out_syn, out_ant)
    rows = (rows0, rows1)
    sems = (sem0, sem1)

    def fire(t):
        cp = pltpu.make_async_copy(
            table_hbm.at[idx_v.at[t]], rows[t % 2], sems[t % 2]
        )
        cp.start()
        return cp

    cp = fire(0)
    for t in range(_NT):
        cp.wait()
        if t + 1 < _NT:
            nxt = fire(t + 1)
        src = rows[t % 2]
        out_hbm = outs[t // _NCHUNK]
        off = base + (t % _NCHUNK) * _CHUNK

        @pl.loop(0, _CHUNK)
        def _repack(k):
            for j in range(18):
                buf[k, pl.ds(j * 16, 16)] = src[k, pl.ds(j * 16, 16)]
            buf[k, pl.ds(284, 16)] = src[k, pl.ds(284, 16)]

        pltpu.sync_copy(buf, out_hbm.at[pl.ds(off, _CHUNK)])
        if t + 1 < _NT:
            cp = nxt


def kernel(target_word, synonym, antonym, embedding_weight):
    tw = target_word.astype(jnp.int32).reshape(_NW, _NCHUNK, _CHUNK)
    syn = synonym.astype(jnp.int32).reshape(_NW, _NCHUNK, _CHUNK)
    ant = antonym.astype(jnp.int32).reshape(_NW, _NCHUNK, _CHUNK)
    tab = _pad_table(embedding_weight)
    o = _embed3(tw, syn, ant, tab)
    return (o[0], o[1], o[2])
